# Initial kernel scaffold; baseline (speedup 1.0000x reference)
import jax
import jax.numpy as jnp
from jax.experimental import pallas as pl


def kernel(input, embed):
    raise NotImplementedError("write your pallas kernel here")


# speedup vs baseline: 1.2565x; 1.2565x over previous
import jax
import jax.numpy as jnp
from jax.experimental import pallas as pl
from jax.experimental.pallas import tpu as pltpu
from jax.experimental.pallas import tpu_sc as plsc

_DIM = 256
_K = 8192
_N_IMG = 16
_HW = 1024
_TOK = _N_IMG * _HW
_T = 256
_NT = _TOK // _T
_ET_ROWS = _K // _NT
_E_W = 0.25


def _argmin_body(x_ref, e_ref, idx_ref, et_ref, esq_ref):
    step = pl.program_id(0)

    @pl.when(step == 0)
    def _():
        esq_ref[...] = jnp.sum(e_ref[...] * e_ref[...], axis=0, keepdims=True)

    et_ref[...] = e_ref[:, pl.ds(step * _ET_ROWS, _ET_ROWS)].T

    x = x_ref[0]
    xt = x.T
    m = jax.lax.dot_general(
        xt, e_ref[...], (((1,), (0,)), ((), ())),
        preferred_element_type=jnp.float32,
    )
    xsq = jnp.sum(xt * xt, axis=1, keepdims=True)
    dist = (xsq - 2.0 * m) + esq_ref[...]
    minval = jnp.min(dist, axis=1, keepdims=True)
    iota = jax.lax.broadcasted_iota(jnp.int32, dist.shape, 1)
    idx = jnp.min(jnp.where(dist == minval, iota, jnp.int32(_K)), axis=1)
    idx_ref[0, 0, :] = idx


def _out_body(q_ref, x_ref, o_ref, l_ref):
    q = q_ref[0]
    qt = q.T
    x = x_ref[0]
    qst = x + (qt - x)
    o_ref[0] = qst
    d = qst - x
    l_ref[0] = jnp.sum(d * d, axis=(0, 1), keepdims=True)


def kernel(input, embed):
    x_cn = input.reshape(_N_IMG, _DIM, _HW)

    idx3, et = pl.pallas_call(
        _argmin_body,
        grid=(_NT,),
        in_specs=[
            pl.BlockSpec((1, _DIM, _T), lambda i: (i // (_HW // _T), 0, i % (_HW // _T))),
            pl.BlockSpec((_DIM, _K), lambda i: (0, 0)),
        ],
        out_specs=[
            pl.BlockSpec((1, 1, _T), lambda i: (i, 0, 0)),
            pl.BlockSpec((_ET_ROWS, _DIM), lambda i: (i, 0)),
        ],
        out_shape=[
            jax.ShapeDtypeStruct((_NT, 1, _T), jnp.int32),
            jax.ShapeDtypeStruct((_K, _DIM), jnp.float32),
        ],
        scratch_shapes=[pltpu.VMEM((1, _K), jnp.float32)],
    )(x_cn, embed)

    idx2 = idx3.reshape(1, _TOK)

    gw = 128
    vector_mesh = plsc.VectorSubcoreMesh(
        core_axis_name="core", subcore_axis_name="subcore"
    )

    @pl.kernel(
        out_type=jax.ShapeDtypeStruct((_TOK, _DIM), jnp.float32),
        mesh=vector_mesh,
    )
    def _gather(et_hbm, i_hbm, o_hbm):
        def body(i_vmem, o_vmem):
            pltpu.sync_copy(et_hbm.at[i_vmem.at[0]], o_vmem)

        pltpu.emit_pipeline(
            body,
            grid=(_TOK // gw,),
            in_specs=[pl.BlockSpec((1, gw), index_map=lambda i: (0, i))],
            out_specs=[pl.BlockSpec((gw, _DIM), index_map=lambda i: (i, 0))],
            core_axis_name=("core", "subcore"),
            dimension_semantics=(pltpu.PARALLEL,),
        )(i_hbm, o_hbm)

    q = _gather(et, idx2)

    q3 = q.reshape(_N_IMG, _HW, _DIM)
    out3, lparts = pl.pallas_call(
        _out_body,
        grid=(_N_IMG,),
        in_specs=[
            pl.BlockSpec((1, _HW, _DIM), lambda n: (n, 0, 0)),
            pl.BlockSpec((1, _DIM, _HW), lambda n: (n, 0, 0)),
        ],
        out_specs=[
            pl.BlockSpec((1, _DIM, _HW), lambda n: (n, 0, 0)),
            pl.BlockSpec((1, 1, 1), lambda n: (n, 0, 0)),
        ],
        out_shape=[
            jax.ShapeDtypeStruct((_N_IMG, _DIM, _HW), jnp.float32),
            jax.ShapeDtypeStruct((_N_IMG, 1, 1), jnp.float32),
        ],
    )(q3, x_cn)

    mse = jnp.sum(lparts) / jnp.float32(_TOK * _DIM)
    out = out3.reshape(_N_IMG, _DIM, 32, 32)
    return (_E_W * mse, out, mse)
